# initial kernel scaffold (unmeasured)
import jax
import jax.numpy as jnp
from jax import lax
from jax.experimental import pallas as pl
from jax.experimental.pallas import tpu as pltpu

N_DEV = 16


def kernel(x, w_mat):
    m, k_per = x.shape
    k, n = w_mat.shape
    m_blk = m // N_DEV

    def body(x_ref, w_ref, out_ref, xb_ref, wb_ref, xg_ref, amax_ref,
             send_sems, recv_sems, send2_sems, recv2_sems):
        my_i = lax.axis_index("i")

        barrier_sem = pltpu.get_barrier_semaphore()
        for d in range(1, N_DEV):
            pl.semaphore_signal(
                barrier_sem, inc=1,
                device_id=((my_i + d) % N_DEV,),
                device_id_type=pl.DeviceIdType.MESH,
            )
        pl.semaphore_wait(barrier_sem, N_DEV - 1)

        xb_ref[:, :] = x_ref[:, :].astype(jnp.bfloat16)

        a2a = []
        for d in range(1, N_DEV):
            t = (my_i + d) % N_DEV
            rdma = pltpu.make_async_remote_copy(
                src_ref=xb_ref.at[pl.ds(t * m_blk, m_blk), :],
                dst_ref=xg_ref.at[d],
                send_sem=send_sems.at[d],
                recv_sem=recv_sems.at[d],
                device_id=(t,),
                device_id_type=pl.DeviceIdType.MESH,
            )
            rdma.start()
            a2a.append(rdma)

        wb_ref[:, :] = w_ref[:, :].astype(jnp.bfloat16)

        out_ref[:, :] = jnp.dot(
            xb_ref[pl.ds(my_i * m_blk, m_blk), :],
            wb_ref[pl.ds(my_i * k_per, k_per), :],
            preferred_element_type=jnp.float32,
        )

        for d in range(1, N_DEV):
            a2a[d - 1].wait()
            s = (my_i - d) % N_DEV
            out_ref[:, :] = out_ref[:, :] + jnp.dot(
                xg_ref[d],
                wb_ref[pl.ds(s * k_per, k_per), :],
                preferred_element_type=jnp.float32,
            )

        local_amax = jnp.max(jnp.abs(out_ref[:, :]))
        amax_ref[0, :] = jnp.full((128,), local_amax, jnp.float32)
        ax = []
        for d in range(1, N_DEV):
            t = (my_i + d) % N_DEV
            rdma = pltpu.make_async_remote_copy(
                src_ref=amax_ref.at[0],
                dst_ref=amax_ref.at[d],
                send_sem=send2_sems.at[d],
                recv_sem=recv2_sems.at[d],
                device_id=(t,),
                device_id_type=pl.DeviceIdType.MESH,
            )
            rdma.start()
            ax.append(rdma)
        for rdma in ax:
            rdma.wait()

        gmax = jnp.max(amax_ref[:, :])
        scale = gmax / 127.0
        q = jnp.clip(jnp.round(out_ref[:, :] / scale), -127.0, 127.0)
        out_ref[:, :] = q * scale

    return pl.pallas_call(
        body,
        out_shape=jax.ShapeDtypeStruct((m_blk, n), jnp.float32),
        in_specs=[
            pl.BlockSpec(memory_space=pltpu.VMEM),
            pl.BlockSpec(memory_space=pltpu.VMEM),
        ],
        out_specs=pl.BlockSpec(memory_space=pltpu.VMEM),
        scratch_shapes=[
            pltpu.VMEM((m, k_per), jnp.bfloat16),
            pltpu.VMEM((k, n), jnp.bfloat16),
            pltpu.VMEM((N_DEV, m_blk, k_per), jnp.bfloat16),
            pltpu.VMEM((N_DEV, 128), jnp.float32),
            pltpu.SemaphoreType.DMA((N_DEV,)),
            pltpu.SemaphoreType.DMA((N_DEV,)),
            pltpu.SemaphoreType.DMA((N_DEV,)),
            pltpu.SemaphoreType.DMA((N_DEV,)),
        ],
        compiler_params=pltpu.CompilerParams(collective_id=0),
    )(x, w_mat)


# baseline (device time: 55569 ns/iter reference)
import jax
import jax.numpy as jnp
from jax import lax
from jax.experimental import pallas as pl
from jax.experimental.pallas import tpu as pltpu

N_DEV = 16


def kernel(x, w_mat):
    m, k_per = x.shape
    k, n = w_mat.shape
    m_blk = m // N_DEV

    def body(x_ref, w_ref, out_ref, xb_ref, wb_ref, xg_ref, amax_ref,
             send_sems, recv_sems, send2_sems, recv2_sems):
        my_i = lax.axis_index("i")

        barrier_sem = pltpu.get_barrier_semaphore()
        for d in range(1, N_DEV):
            pl.semaphore_signal(
                barrier_sem, inc=1,
                device_id=((my_i + d) % N_DEV,),
                device_id_type=pl.DeviceIdType.MESH,
            )
        pl.semaphore_wait(barrier_sem, N_DEV - 1)

        xb_ref[:, :] = x_ref[:, :].astype(jnp.bfloat16)

        a2a = []
        for d in range(1, N_DEV):
            t = (my_i + d) % N_DEV
            rdma = pltpu.make_async_remote_copy(
                src_ref=xb_ref.at[pl.ds(t * m_blk, m_blk), :],
                dst_ref=xg_ref.at[d],
                send_sem=send_sems.at[d],
                recv_sem=recv_sems.at[d],
                device_id=(t,),
                device_id_type=pl.DeviceIdType.MESH,
            )
            rdma.start()
            a2a.append(rdma)

        wb_ref[:, :] = w_ref[:, :].astype(jnp.bfloat16)

        out_ref[:, :] = jnp.dot(
            xb_ref[pl.ds(my_i * m_blk, m_blk), :],
            wb_ref[pl.ds(my_i * k_per, k_per), :],
            preferred_element_type=jnp.float32,
        )

        for d in range(1, N_DEV):
            a2a[d - 1].wait()
            s = (my_i - d) % N_DEV
            out_ref[:, :] = out_ref[:, :] + jnp.dot(
                xg_ref[d],
                wb_ref[pl.ds(s * k_per, k_per), :],
                preferred_element_type=jnp.float32,
            )

        local_amax = jnp.max(jnp.abs(out_ref[:, :]))
        amax_ref[0, :] = jnp.full((128,), local_amax, jnp.float32)
        ax = []
        for d in range(1, N_DEV):
            t = (my_i + d) % N_DEV
            rdma = pltpu.make_async_remote_copy(
                src_ref=amax_ref.at[0],
                dst_ref=amax_ref.at[d],
                send_sem=send2_sems.at[d],
                recv_sem=recv2_sems.at[d],
                device_id=(t,),
                device_id_type=pl.DeviceIdType.MESH,
            )
            rdma.start()
            ax.append(rdma)
        for rdma in ax:
            rdma.wait()

        gmax = jnp.max(amax_ref[:, :])
        scale = gmax / 127.0
        q = jnp.clip(jnp.round(out_ref[:, :] / scale), -127.0, 127.0)
        out_ref[:, :] = q * scale

    return pl.pallas_call(
        body,
        out_shape=jax.ShapeDtypeStruct((m_blk, n), jnp.float32),
        in_specs=[
            pl.BlockSpec(memory_space=pltpu.VMEM),
            pl.BlockSpec(memory_space=pltpu.VMEM),
        ],
        out_specs=pl.BlockSpec(memory_space=pltpu.VMEM),
        scratch_shapes=[
            pltpu.VMEM((m, k_per), jnp.bfloat16),
            pltpu.VMEM((k, n), jnp.bfloat16),
            pltpu.VMEM((N_DEV, m_blk, k_per), jnp.bfloat16),
            pltpu.VMEM((N_DEV, 128), jnp.float32),
            pltpu.SemaphoreType.DMA((N_DEV,)),
            pltpu.SemaphoreType.DMA((N_DEV,)),
            pltpu.SemaphoreType.DMA((N_DEV,)),
            pltpu.SemaphoreType.DMA((N_DEV,)),
        ],
        compiler_params=pltpu.CompilerParams(
            collective_id=0,
            vmem_limit_bytes=100 * 1024 * 1024,
        ),
    )(x, w_mat)


# device time: 53493 ns/iter; 1.0388x vs baseline; 1.0388x over previous
import os

import jax
import jax.numpy as jnp
from jax import lax
from jax.experimental import pallas as pl
from jax.experimental.pallas import tpu as pltpu

N_DEV = 16

_ABLATE = os.environ.get("ABLATE", "")


def kernel(x, w_mat):
    m, k_per = x.shape
    k, n = w_mat.shape
    m_blk = m // N_DEV

    def body(x_ref, w_ref, out_ref, xb_ref, wb_ref, xg_ref, amax_ref,
             send_sems, recv_sems, send2_sems, recv2_sems):
        my_i = lax.axis_index("i")

        if _ABLATE != "compute":
            barrier_sem = pltpu.get_barrier_semaphore()
            for d in range(1, N_DEV):
                pl.semaphore_signal(
                    barrier_sem, inc=1,
                    device_id=((my_i + d) % N_DEV,),
                    device_id_type=pl.DeviceIdType.MESH,
                )
            pl.semaphore_wait(barrier_sem, N_DEV - 1)

        xb_ref[:, :] = x_ref[:, :].astype(jnp.bfloat16)

        a2a = []
        if _ABLATE != "compute":
            for d in range(1, N_DEV):
                t = (my_i + d) % N_DEV
                rdma = pltpu.make_async_remote_copy(
                    src_ref=xb_ref.at[pl.ds(t * m_blk, m_blk), :],
                    dst_ref=xg_ref.at[d],
                    send_sem=send_sems.at[d],
                    recv_sem=recv_sems.at[d],
                    device_id=(t,),
                    device_id_type=pl.DeviceIdType.MESH,
                )
                rdma.start()
                a2a.append(rdma)

        wb_ref[:, :] = w_ref[:, :].astype(jnp.bfloat16)

        out_ref[:, :] = jnp.dot(
            xb_ref[pl.ds(my_i * m_blk, m_blk), :],
            wb_ref[pl.ds(my_i * k_per, k_per), :],
            preferred_element_type=jnp.float32,
        )

        for d in range(1, N_DEV):
            if _ABLATE != "compute":
                a2a[d - 1].wait()
            if _ABLATE != "comm":
                s = (my_i - d) % N_DEV
                out_ref[:, :] = out_ref[:, :] + jnp.dot(
                    xg_ref[d],
                    wb_ref[pl.ds(s * k_per, k_per), :],
                    preferred_element_type=jnp.float32,
                )

        local_amax = jnp.max(jnp.abs(out_ref[:, :]))
        amax_ref[0, :] = jnp.full((128,), local_amax, jnp.float32)
        if _ABLATE != "compute":
            ax = []
            for d in range(1, N_DEV):
                t = (my_i + d) % N_DEV
                rdma = pltpu.make_async_remote_copy(
                    src_ref=amax_ref.at[0],
                    dst_ref=amax_ref.at[d],
                    send_sem=send2_sems.at[d],
                    recv_sem=recv2_sems.at[d],
                    device_id=(t,),
                    device_id_type=pl.DeviceIdType.MESH,
                )
                rdma.start()
                ax.append(rdma)
            for rdma in ax:
                rdma.wait()

        gmax = jnp.max(amax_ref[:, :])
        scale = gmax / 127.0
        q = jnp.clip(jnp.round(out_ref[:, :] / scale), -127.0, 127.0)
        out_ref[:, :] = q * scale

    return pl.pallas_call(
        body,
        out_shape=jax.ShapeDtypeStruct((m_blk, n), jnp.float32),
        in_specs=[
            pl.BlockSpec(memory_space=pltpu.VMEM),
            pl.BlockSpec(memory_space=pltpu.VMEM),
        ],
        out_specs=pl.BlockSpec(memory_space=pltpu.VMEM),
        scratch_shapes=[
            pltpu.VMEM((m, k_per), jnp.bfloat16),
            pltpu.VMEM((k, n), jnp.bfloat16),
            pltpu.VMEM((N_DEV, m_blk, k_per), jnp.bfloat16),
            pltpu.VMEM((N_DEV, 128), jnp.float32),
            pltpu.SemaphoreType.DMA((N_DEV,)),
            pltpu.SemaphoreType.DMA((N_DEV,)),
            pltpu.SemaphoreType.DMA((N_DEV,)),
            pltpu.SemaphoreType.DMA((N_DEV,)),
        ],
        compiler_params=pltpu.CompilerParams(
            collective_id=None if _ABLATE == "compute" else 0,
            vmem_limit_bytes=100 * 1024 * 1024,
        ),
    )(x, w_mat)


# device time: 32129 ns/iter; 1.7296x vs baseline; 1.6649x over previous
import os

import jax
import jax.numpy as jnp
from jax import lax
from jax.experimental import pallas as pl
from jax.experimental.pallas import tpu as pltpu

N_DEV = 16

_ABLATE = os.environ.get("ABLATE", "")
_NFLOWS = int(os.environ.get("NFLOWS", str(N_DEV - 1)))


def kernel(x, w_mat):
    m, k_per = x.shape
    k, n = w_mat.shape
    m_blk = m // N_DEV

    def body(x_ref, w_ref, out_ref, xb_ref, wb_ref, xg_ref, amax_ref,
             send_sems, recv_sems, send2_sems, recv2_sems):
        my_i = lax.axis_index("i")

        if _ABLATE != "compute":
            barrier_sem = pltpu.get_barrier_semaphore()
            for d in range(1, N_DEV):
                pl.semaphore_signal(
                    barrier_sem, inc=1,
                    device_id=((my_i + d) % N_DEV,),
                    device_id_type=pl.DeviceIdType.MESH,
                )
            pl.semaphore_wait(barrier_sem, N_DEV - 1)

        xb_ref[:, :] = x_ref[:, :].astype(jnp.bfloat16)

        a2a = []
        if _ABLATE != "compute":
            for d in range(1, 1 + _NFLOWS):
                t = (my_i + d) % N_DEV
                rdma = pltpu.make_async_remote_copy(
                    src_ref=xb_ref.at[pl.ds(t * m_blk, m_blk), :],
                    dst_ref=xg_ref.at[d],
                    send_sem=send_sems.at[d],
                    recv_sem=recv_sems.at[d],
                    device_id=(t,),
                    device_id_type=pl.DeviceIdType.MESH,
                )
                rdma.start()
                a2a.append(rdma)

        wb_ref[:, :] = w_ref[:, :].astype(jnp.bfloat16)

        out_ref[:, :] = jnp.dot(
            xb_ref[pl.ds(my_i * m_blk, m_blk), :],
            wb_ref[pl.ds(my_i * k_per, k_per), :],
            preferred_element_type=jnp.float32,
        )

        for d in range(1, N_DEV):
            if _ABLATE != "compute" and d <= _NFLOWS:
                a2a[d - 1].wait()
            if _ABLATE != "comm":
                s = (my_i - d) % N_DEV
                out_ref[:, :] = out_ref[:, :] + jnp.dot(
                    xg_ref[d],
                    wb_ref[pl.ds(s * k_per, k_per), :],
                    preferred_element_type=jnp.float32,
                )

        local_amax = jnp.max(jnp.abs(out_ref[:, :]))
        amax_ref[0, :] = jnp.full((128,), local_amax, jnp.float32)
        if _ABLATE != "compute":
            ax = []
            for d in range(1, N_DEV):
                t = (my_i + d) % N_DEV
                rdma = pltpu.make_async_remote_copy(
                    src_ref=amax_ref.at[0],
                    dst_ref=amax_ref.at[d],
                    send_sem=send2_sems.at[d],
                    recv_sem=recv2_sems.at[d],
                    device_id=(t,),
                    device_id_type=pl.DeviceIdType.MESH,
                )
                rdma.start()
                ax.append(rdma)
            for rdma in ax:
                rdma.wait()

        gmax = jnp.max(amax_ref[:, :])
        scale = gmax / 127.0
        q = jnp.clip(jnp.round(out_ref[:, :] / scale), -127.0, 127.0)
        out_ref[:, :] = q * scale

    return pl.pallas_call(
        body,
        out_shape=jax.ShapeDtypeStruct((m_blk, n), jnp.float32),
        in_specs=[
            pl.BlockSpec(memory_space=pltpu.VMEM),
            pl.BlockSpec(memory_space=pltpu.VMEM),
        ],
        out_specs=pl.BlockSpec(memory_space=pltpu.VMEM),
        scratch_shapes=[
            pltpu.VMEM((m, k_per), jnp.bfloat16),
            pltpu.VMEM((k, n), jnp.bfloat16),
            pltpu.VMEM((N_DEV, m_blk, k_per), jnp.bfloat16),
            pltpu.VMEM((N_DEV, 128), jnp.float32),
            pltpu.SemaphoreType.DMA((N_DEV,)),
            pltpu.SemaphoreType.DMA((N_DEV,)),
            pltpu.SemaphoreType.DMA((N_DEV,)),
            pltpu.SemaphoreType.DMA((N_DEV,)),
        ],
        compiler_params=pltpu.CompilerParams(
            collective_id=None if _ABLATE == "compute" else 0,
            vmem_limit_bytes=100 * 1024 * 1024,
        ),
    )(x, w_mat)
